# initial kernel scaffold (unmeasured)
import jax
import jax.numpy as jnp
from jax import lax
from jax.experimental import pallas as pl
from jax.experimental.pallas import tpu as pltpu


def kernel(
    x,
):
    def body(*refs):
        pass

    out_shape = jax.ShapeDtypeStruct(..., jnp.float32)
    return pl.pallas_call(body, out_shape=out_shape)(...)


# baseline (device time: 1187716 ns/iter reference)
import jax
import jax.numpy as jnp
from jax import lax
from jax.experimental import pallas as pl
from jax.experimental.pallas import tpu as pltpu

N_DEV = 4
LOG_M_PER = 12
M_PER = 1 << LOG_M_PER
LOG_M_TOT = 14
M_TOT = N_DEV * M_PER
N_COLS = 1024

_TILE = 2048
_SORT_CHUNK = 256
_MERGE_CHUNK = 128

_VMEM_LIMIT = pltpu.CompilerParams(vmem_limit_bytes=60 * 1024 * 1024)


def _cx_tiles(read, write, m, j, k, desc_flip=None):
    d = 1 << j
    if d >= _TILE:
        for t in range(0, m, _TILE):
            first = ((t >> j) & 1) == 0
            pt = t + d if first else t - d
            a = read(t)
            b = read(pt)
            desc = ((t >> k) & 1) == 1
            take_min = first != desc
            if desc_flip is None:
                r = jnp.minimum(a, b) if take_min else jnp.maximum(a, b)
            else:
                lo, hi = jnp.minimum(a, b), jnp.maximum(a, b)
                r = jnp.where(
                    jnp.logical_xor(take_min, desc_flip), lo, hi
                )
            write(t, r)
    else:
        iota = lax.broadcasted_iota(jnp.int32, (_TILE, 1), 0)
        is_first = ((iota >> j) & 1) == 0
        for t in range(0, m, _TILE):
            x = read(t)
            up = jnp.roll(x, -d, axis=0)
            down = jnp.roll(x, d, axis=0)
            partner = jnp.where(is_first, up, down)
            if (1 << k) >= _TILE:
                desc = ((t >> k) & 1) == 1
                take_min = jnp.logical_not(is_first) if desc else is_first
            else:
                descv = ((iota >> k) & 1) == 1
                take_min = jnp.logical_xor(is_first, descv)
            if desc_flip is not None:
                take_min = jnp.logical_xor(take_min, desc_flip)
            r = jnp.where(
                take_min, jnp.minimum(x, partner), jnp.maximum(x, partner)
            )
            write(t, r)


def _reader(ref, off=None):
    if off is None:
        return lambda t: ref[pl.ds(t, _TILE), :]
    return lambda t: ref[pl.ds(off + t, _TILE), :]


def _writer(ref):
    def w(t, v):
        ref[pl.ds(t, _TILE), :] = v
    return w


def _local_sort_body(x_ref, o_ref, s0, s1):
    p = lax.axis_index("i")
    odd = lax.rem(p, 2) == 1
    subs = [
        (j, k)
        for k in range(1, LOG_M_PER + 1)
        for j in range(k - 1, -1, -1)
    ]
    bufs = [s0, s1]
    for idx, (j, k) in enumerate(subs):
        if idx == 0:
            read = lambda t: x_ref[pl.ds(t, _TILE), :].astype(jnp.bfloat16)
        else:
            read = _reader(bufs[(idx - 1) % 2])
        dst = o_ref if idx == len(subs) - 1 else bufs[idx % 2]
        flip = odd if k == LOG_M_PER else None
        _cx_tiles(read, _writer(dst), M_PER, j, k, desc_flip=flip)


def _local_sort(x):
    m, n = x.shape
    grid = n // _SORT_CHUNK
    return pl.pallas_call(
        _local_sort_body,
        grid=(grid,),
        in_specs=[pl.BlockSpec((m, _SORT_CHUNK), lambda c: (0, c))],
        out_specs=pl.BlockSpec((m, _SORT_CHUNK), lambda c: (0, c)),
        out_shape=jax.ShapeDtypeStruct((m, n), jnp.bfloat16),
        scratch_shapes=[
            pltpu.VMEM((M_PER, _SORT_CHUNK), jnp.bfloat16),
            pltpu.VMEM((M_PER, _SORT_CHUNK), jnp.bfloat16),
        ],
        compiler_params=_VMEM_LIMIT,
    )(x)


def _gather_body(x_ref, o_ref, comm_ref, send_sems, recv_sems, copy_sem):
    p = lax.axis_index("i")
    right = lax.rem(p + 1, N_DEV)
    left = lax.rem(p - 1 + N_DEV, N_DEV)

    barrier_sem = pltpu.get_barrier_semaphore()
    for nbr in (left, right):
        pl.semaphore_signal(
            barrier_sem, inc=1,
            device_id=(nbr,), device_id_type=pl.DeviceIdType.MESH,
        )
    pl.semaphore_wait(barrier_sem, 2)

    comm_ref[0] = x_ref[...]
    own = pltpu.make_async_copy(
        comm_ref.at[0], o_ref.at[pl.ds(p * M_PER, M_PER), :], copy_sem
    )
    own.start()
    own.wait()

    for h in range(N_DEV - 1):
        send_slot = h % 2
        recv_slot = (h + 1) % 2
        rdma = pltpu.make_async_remote_copy(
            src_ref=comm_ref.at[send_slot],
            dst_ref=comm_ref.at[recv_slot],
            send_sem=send_sems.at[send_slot],
            recv_sem=recv_sems.at[recv_slot],
            device_id=(right,),
            device_id_type=pl.DeviceIdType.MESH,
        )
        rdma.start()
        rdma.wait()

        origin = lax.rem(p - (h + 1) + N_DEV, N_DEV)
        store = pltpu.make_async_copy(
            comm_ref.at[recv_slot],
            o_ref.at[pl.ds(origin * M_PER, M_PER), :],
            copy_sem,
        )
        store.start()
        store.wait()


def _all_gather(xs):
    m, n = xs.shape
    return pl.pallas_call(
        _gather_body,
        in_specs=[pl.BlockSpec(memory_space=pltpu.VMEM)],
        out_specs=pl.BlockSpec(memory_space=pl.ANY),
        out_shape=jax.ShapeDtypeStruct((N_DEV * m, n), jnp.bfloat16),
        scratch_shapes=[
            pltpu.VMEM((2, m, n), jnp.bfloat16),
            pltpu.SemaphoreType.DMA((2,)),
            pltpu.SemaphoreType.DMA((2,)),
            pltpu.SemaphoreType.DMA,
        ],
        compiler_params=pltpu.CompilerParams(collective_id=0),
    )(xs)


def _merge_body(g_ref, o_ref, s0, s1):
    p = lax.axis_index("i")
    bufs = [s0, s1]
    idx = 0

    def step(read, height, j, k, last=False):
        nonlocal idx
        dst = o_ref if last else bufs[idx % 2]
        _cx_tiles(read, _writer(dst), height, j, k)
        idx += 1

    def src():
        return bufs[(idx - 1) % 2]

    step(_reader(g_ref), M_TOT, LOG_M_TOT - 2, LOG_M_TOT - 1)
    for j in range(LOG_M_TOT - 3, -1, -1):
        step(_reader(src()), M_TOT, j, LOG_M_TOT - 1)

    step(_reader(src()), M_TOT, LOG_M_TOT - 1, LOG_M_TOT)
    base8 = lax.div(p, 2) * (M_TOT // 2)
    step(_reader(src(), base8), M_TOT // 2, LOG_M_TOT - 2, LOG_M_TOT)
    base4 = lax.rem(p, 2) * M_PER
    step(_reader(src(), base4), M_PER, LOG_M_PER - 1, LOG_M_TOT)
    for j in range(LOG_M_PER - 2, -1, -1):
        step(_reader(src()), M_PER, j, LOG_M_TOT, last=(j == 0))


def _merge(g):
    grid = N_COLS // _MERGE_CHUNK
    return pl.pallas_call(
        _merge_body,
        grid=(grid,),
        in_specs=[pl.BlockSpec((M_TOT, _MERGE_CHUNK), lambda c: (0, c))],
        out_specs=pl.BlockSpec((M_PER, _MERGE_CHUNK), lambda c: (0, c)),
        out_shape=jax.ShapeDtypeStruct((M_PER, N_COLS), jnp.bfloat16),
        scratch_shapes=[
            pltpu.VMEM((M_TOT, _MERGE_CHUNK), jnp.bfloat16),
            pltpu.VMEM((M_TOT, _MERGE_CHUNK), jnp.bfloat16),
        ],
        compiler_params=_VMEM_LIMIT,
    )(g)


def kernel(x):
    xs = _local_sort(x)
    g = _all_gather(xs)
    return _merge(g)


# device time: 1035306 ns/iter; 1.1472x vs baseline; 1.1472x over previous
import jax
import jax.numpy as jnp
from jax import lax
from jax.experimental import pallas as pl
from jax.experimental.pallas import tpu as pltpu

N_DEV = 4
LOG_M_PER = 12
M_PER = 1 << LOG_M_PER
LOG_M_TOT = 14
M_TOT = N_DEV * M_PER
N_COLS = 1024

_TILE = 2048
_SORT_CHUNK = 256
_MERGE_CHUNK = 128

_VMEM_LIMIT = pltpu.CompilerParams(vmem_limit_bytes=60 * 1024 * 1024)


_STATIC_MIN = 512


def _cx_tiles(read, write, m, j, k, desc_flip=None):
    d = 1 << j
    if d >= _STATIC_MIN:
        ts = min(_TILE, d)
        for t in range(0, m, ts):
            first = ((t >> j) & 1) == 0
            pt = t + d if first else t - d
            a = read(t, ts)
            b = read(pt, ts)
            desc = ((t >> k) & 1) == 1
            take_min = first != desc
            if desc_flip is None:
                r = jnp.minimum(a, b) if take_min else jnp.maximum(a, b)
            else:
                lo, hi = jnp.minimum(a, b), jnp.maximum(a, b)
                r = jnp.where(
                    jnp.logical_xor(take_min, desc_flip), lo, hi
                )
            write(t, r, ts)
    else:
        iota = lax.broadcasted_iota(jnp.int32, (_TILE, 1), 0)
        is_first = ((iota >> j) & 1) == 0
        uniform = (1 << k) >= _TILE
        for t in range(0, m, _TILE):
            x = read(t, _TILE)
            up = jnp.roll(x, -d, axis=0)
            down = jnp.roll(x, d, axis=0)
            if uniform and desc_flip is None:
                if ((t >> k) & 1) == 1:
                    r = jnp.where(is_first, jnp.maximum(x, up),
                                  jnp.minimum(x, down))
                else:
                    r = jnp.where(is_first, jnp.minimum(x, up),
                                  jnp.maximum(x, down))
            else:
                partner = jnp.where(is_first, up, down)
                if uniform:
                    desc = ((t >> k) & 1) == 1
                    take_min = jnp.logical_not(is_first) if desc else is_first
                else:
                    descv = ((iota >> k) & 1) == 1
                    take_min = jnp.logical_xor(is_first, descv)
                if desc_flip is not None:
                    take_min = jnp.logical_xor(take_min, desc_flip)
                r = jnp.where(
                    take_min, jnp.minimum(x, partner), jnp.maximum(x, partner)
                )
            write(t, r, _TILE)


def _reader(ref, off=None):
    if off is None:
        return lambda t, h: ref[pl.ds(t, h), :]
    return lambda t, h: ref[pl.ds(off + t, h), :]


def _writer(ref):
    def w(t, v, h):
        ref[pl.ds(t, h), :] = v
    return w


def _local_sort_body(x_ref, o_ref, s0, s1):
    p = lax.axis_index("i")
    odd = lax.rem(p, 2) == 1
    subs = [
        (j, k)
        for k in range(1, LOG_M_PER + 1)
        for j in range(k - 1, -1, -1)
    ]
    bufs = [s0, s1]
    for idx, (j, k) in enumerate(subs):
        if idx == 0:
            read = lambda t, h: x_ref[pl.ds(t, h), :].astype(jnp.bfloat16)
        else:
            read = _reader(bufs[(idx - 1) % 2])
        dst = o_ref if idx == len(subs) - 1 else bufs[idx % 2]
        flip = odd if k == LOG_M_PER else None
        _cx_tiles(read, _writer(dst), M_PER, j, k, desc_flip=flip)


def _local_sort(x):
    m, n = x.shape
    grid = n // _SORT_CHUNK
    return pl.pallas_call(
        _local_sort_body,
        grid=(grid,),
        in_specs=[pl.BlockSpec((m, _SORT_CHUNK), lambda c: (0, c))],
        out_specs=pl.BlockSpec((m, _SORT_CHUNK), lambda c: (0, c)),
        out_shape=jax.ShapeDtypeStruct((m, n), jnp.bfloat16),
        scratch_shapes=[
            pltpu.VMEM((M_PER, _SORT_CHUNK), jnp.bfloat16),
            pltpu.VMEM((M_PER, _SORT_CHUNK), jnp.bfloat16),
        ],
        compiler_params=_VMEM_LIMIT,
    )(x)


def _gather_body(x_ref, o_ref, comm_ref, send_sems, recv_sems, copy_sem):
    p = lax.axis_index("i")
    right = lax.rem(p + 1, N_DEV)
    left = lax.rem(p - 1 + N_DEV, N_DEV)

    barrier_sem = pltpu.get_barrier_semaphore()
    for nbr in (left, right):
        pl.semaphore_signal(
            barrier_sem, inc=1,
            device_id=(nbr,), device_id_type=pl.DeviceIdType.MESH,
        )
    pl.semaphore_wait(barrier_sem, 2)

    comm_ref[0] = x_ref[...]
    own = pltpu.make_async_copy(
        comm_ref.at[0], o_ref.at[pl.ds(p * M_PER, M_PER), :], copy_sem
    )
    own.start()
    own.wait()

    for h in range(N_DEV - 1):
        send_slot = h % 2
        recv_slot = (h + 1) % 2
        rdma = pltpu.make_async_remote_copy(
            src_ref=comm_ref.at[send_slot],
            dst_ref=comm_ref.at[recv_slot],
            send_sem=send_sems.at[send_slot],
            recv_sem=recv_sems.at[recv_slot],
            device_id=(right,),
            device_id_type=pl.DeviceIdType.MESH,
        )
        rdma.start()
        rdma.wait()

        origin = lax.rem(p - (h + 1) + N_DEV, N_DEV)
        store = pltpu.make_async_copy(
            comm_ref.at[recv_slot],
            o_ref.at[pl.ds(origin * M_PER, M_PER), :],
            copy_sem,
        )
        store.start()
        store.wait()


def _all_gather(xs):
    m, n = xs.shape
    return pl.pallas_call(
        _gather_body,
        in_specs=[pl.BlockSpec(memory_space=pltpu.VMEM)],
        out_specs=pl.BlockSpec(memory_space=pl.ANY),
        out_shape=jax.ShapeDtypeStruct((N_DEV * m, n), jnp.bfloat16),
        scratch_shapes=[
            pltpu.VMEM((2, m, n), jnp.bfloat16),
            pltpu.SemaphoreType.DMA((2,)),
            pltpu.SemaphoreType.DMA((2,)),
            pltpu.SemaphoreType.DMA,
        ],
        compiler_params=pltpu.CompilerParams(collective_id=0),
    )(xs)


def _merge_body(g_ref, o_ref, s0, s1):
    p = lax.axis_index("i")
    bufs = [s0, s1]
    idx = 0

    def step(read, height, j, k, last=False):
        nonlocal idx
        dst = o_ref if last else bufs[idx % 2]
        _cx_tiles(read, _writer(dst), height, j, k)
        idx += 1

    def src():
        return bufs[(idx - 1) % 2]

    step(_reader(g_ref), M_TOT, LOG_M_TOT - 2, LOG_M_TOT - 1)
    for j in range(LOG_M_TOT - 3, -1, -1):
        step(_reader(src()), M_TOT, j, LOG_M_TOT - 1)

    step(_reader(src()), M_TOT, LOG_M_TOT - 1, LOG_M_TOT)
    base8 = lax.div(p, 2) * (M_TOT // 2)
    step(_reader(src(), base8), M_TOT // 2, LOG_M_TOT - 2, LOG_M_TOT)
    base4 = lax.rem(p, 2) * M_PER
    step(_reader(src(), base4), M_PER, LOG_M_PER - 1, LOG_M_TOT)
    for j in range(LOG_M_PER - 2, -1, -1):
        step(_reader(src()), M_PER, j, LOG_M_TOT, last=(j == 0))


def _merge(g):
    grid = N_COLS // _MERGE_CHUNK
    return pl.pallas_call(
        _merge_body,
        grid=(grid,),
        in_specs=[pl.BlockSpec((M_TOT, _MERGE_CHUNK), lambda c: (0, c))],
        out_specs=pl.BlockSpec((M_PER, _MERGE_CHUNK), lambda c: (0, c)),
        out_shape=jax.ShapeDtypeStruct((M_PER, N_COLS), jnp.bfloat16),
        scratch_shapes=[
            pltpu.VMEM((M_TOT, _MERGE_CHUNK), jnp.bfloat16),
            pltpu.VMEM((M_TOT, _MERGE_CHUNK), jnp.bfloat16),
        ],
        compiler_params=_VMEM_LIMIT,
    )(g)


def kernel(x):
    xs = _local_sort(x)
    g = _all_gather(xs)
    return _merge(g)


# device time: 892569 ns/iter; 1.3307x vs baseline; 1.1599x over previous
import functools

import jax
import jax.numpy as jnp
from jax import lax
from jax.experimental import pallas as pl
from jax.experimental.pallas import tpu as pltpu

N_DEV = 4
LOG_M_PER = 12
M_PER = 1 << LOG_M_PER
LOG_M_TOT = 14
M_TOT = N_DEV * M_PER
N_COLS = 1024

_TILE = 2048
_SORT_CHUNK = 256
_MERGE_CHUNK = 128

_VMEM_LIMIT = pltpu.CompilerParams(vmem_limit_bytes=60 * 1024 * 1024)


_STATIC_MIN = 512


def _cx_tiles(read, write, m, j, k, desc_flip=None):
    d = 1 << j
    if d >= _STATIC_MIN:
        ts = min(_TILE, d)
        for t in range(0, m, ts):
            first = ((t >> j) & 1) == 0
            pt = t + d if first else t - d
            a = read(t, ts)
            b = read(pt, ts)
            desc = ((t >> k) & 1) == 1
            take_min = first != desc
            if desc_flip is None:
                r = jnp.minimum(a, b) if take_min else jnp.maximum(a, b)
            else:
                lo, hi = jnp.minimum(a, b), jnp.maximum(a, b)
                r = jnp.where(
                    jnp.logical_xor(take_min, desc_flip), lo, hi
                )
            write(t, r, ts)
    else:
        iota = lax.broadcasted_iota(jnp.int32, (_TILE, 1), 0)
        is_first = ((iota >> j) & 1) == 0
        uniform = (1 << k) >= _TILE
        for t in range(0, m, _TILE):
            x = read(t, _TILE)
            up = jnp.roll(x, -d, axis=0)
            down = jnp.roll(x, d, axis=0)
            if uniform and desc_flip is None:
                if ((t >> k) & 1) == 1:
                    r = jnp.where(is_first, jnp.maximum(x, up),
                                  jnp.minimum(x, down))
                else:
                    r = jnp.where(is_first, jnp.minimum(x, up),
                                  jnp.maximum(x, down))
            else:
                partner = jnp.where(is_first, up, down)
                if uniform:
                    desc = ((t >> k) & 1) == 1
                    take_min = jnp.logical_not(is_first) if desc else is_first
                else:
                    descv = ((iota >> k) & 1) == 1
                    take_min = jnp.logical_xor(is_first, descv)
                if desc_flip is not None:
                    take_min = jnp.logical_xor(take_min, desc_flip)
                r = jnp.where(
                    take_min, jnp.minimum(x, partner), jnp.maximum(x, partner)
                )
            write(t, r, _TILE)


def _reader(ref, off=None):
    if off is None:
        return lambda t, h: ref[pl.ds(t, h), :]
    return lambda t, h: ref[pl.ds(off + t, h), :]


def _writer(ref):
    def w(t, v, h):
        ref[pl.ds(t, h), :] = v
    return w


def _local_sort_body(x_ref, o_ref, s0, s1):
    p = lax.axis_index("i")
    odd = lax.rem(p, 2) == 1
    subs = [
        (j, k)
        for k in range(1, LOG_M_PER + 1)
        for j in range(k - 1, -1, -1)
    ]
    bufs = [s0, s1]
    for idx, (j, k) in enumerate(subs):
        if idx == 0:
            read = lambda t, h: x_ref[pl.ds(t, h), :].astype(jnp.bfloat16)
        else:
            read = _reader(bufs[(idx - 1) % 2])
        dst = o_ref if idx == len(subs) - 1 else bufs[idx % 2]
        flip = odd if k == LOG_M_PER else None
        _cx_tiles(read, _writer(dst), M_PER, j, k, desc_flip=flip)


def _local_sort(x):
    m, n = x.shape
    grid = n // _SORT_CHUNK
    return pl.pallas_call(
        _local_sort_body,
        grid=(grid,),
        in_specs=[pl.BlockSpec((m, _SORT_CHUNK), lambda c: (0, c))],
        out_specs=pl.BlockSpec((m, _SORT_CHUNK), lambda c: (0, c)),
        out_shape=jax.ShapeDtypeStruct((m, n), jnp.bfloat16),
        scratch_shapes=[
            pltpu.VMEM((M_PER, _SORT_CHUNK), jnp.bfloat16),
            pltpu.VMEM((M_PER, _SORT_CHUNK), jnp.bfloat16),
        ],
        compiler_params=_VMEM_LIMIT,
    )(x)


def _exchange_body(x_ref, o_ref, send_sem, recv_sem, copy_sem, *, peer_of,
                   slot_of):
    p = lax.axis_index("i")
    peer = peer_of(p)
    m = x_ref.shape[0]
    off = slot_of(p) * m

    barrier_sem = pltpu.get_barrier_semaphore()
    pl.semaphore_signal(
        barrier_sem, inc=1,
        device_id=(peer,), device_id_type=pl.DeviceIdType.MESH,
    )
    pl.semaphore_wait(barrier_sem, 1)

    rdma = pltpu.make_async_remote_copy(
        src_ref=x_ref,
        dst_ref=o_ref.at[pl.ds(off, m), :],
        send_sem=send_sem,
        recv_sem=recv_sem,
        device_id=(peer,),
        device_id_type=pl.DeviceIdType.MESH,
    )
    rdma.start()
    own = pltpu.make_async_copy(
        x_ref, o_ref.at[pl.ds(off, m), :], copy_sem
    )
    own.start()
    own.wait()
    rdma.wait()


def _exchange(x, peer_of, slot_of, collective_id):
    m, n = x.shape
    body = functools.partial(
        _exchange_body, peer_of=peer_of, slot_of=slot_of
    )
    return pl.pallas_call(
        body,
        in_specs=[pl.BlockSpec(memory_space=pltpu.VMEM)],
        out_specs=pl.BlockSpec(memory_space=pl.ANY),
        out_shape=jax.ShapeDtypeStruct((2 * m, n), jnp.bfloat16),
        scratch_shapes=[
            pltpu.SemaphoreType.DMA,
            pltpu.SemaphoreType.DMA,
            pltpu.SemaphoreType.DMA,
        ],
        compiler_params=pltpu.CompilerParams(collective_id=collective_id),
    )(x)


def _pair_merge_body(pb_ref, o_ref, s0, s1):
    p = lax.axis_index("i")
    pair_odd = lax.div(p, 2) == 1
    bufs = [s0, s1]
    subs = list(range(LOG_M_PER, -1, -1))
    for idx, j in enumerate(subs):
        read = _reader(pb_ref if idx == 0 else bufs[(idx - 1) % 2])
        dst = o_ref if idx == len(subs) - 1 else bufs[idx % 2]
        _cx_tiles(read, _writer(dst), 2 * M_PER, j, LOG_M_TOT - 1,
                  desc_flip=pair_odd)


def _pair_merge(pb):
    m = 2 * M_PER
    grid = N_COLS // _SORT_CHUNK
    return pl.pallas_call(
        _pair_merge_body,
        grid=(grid,),
        in_specs=[pl.BlockSpec((m, _SORT_CHUNK), lambda c: (0, c))],
        out_specs=pl.BlockSpec((m, _SORT_CHUNK), lambda c: (0, c)),
        out_shape=jax.ShapeDtypeStruct((m, N_COLS), jnp.bfloat16),
        scratch_shapes=[
            pltpu.VMEM((m, _SORT_CHUNK), jnp.bfloat16),
            pltpu.VMEM((m, _SORT_CHUNK), jnp.bfloat16),
        ],
        compiler_params=_VMEM_LIMIT,
    )(pb)


def _merge14_body(g_ref, o_ref, s0, s1):
    p = lax.axis_index("i")
    bufs = [s0, s1]
    idx = 0

    def step(read, height, j, k, last=False):
        nonlocal idx
        dst = o_ref if last else bufs[idx % 2]
        _cx_tiles(read, _writer(dst), height, j, k)
        idx += 1

    def src():
        return bufs[(idx - 1) % 2]

    step(_reader(g_ref), M_TOT, LOG_M_TOT - 1, LOG_M_TOT)
    base8 = lax.div(p, 2) * (M_TOT // 2)
    step(_reader(src(), base8), M_TOT // 2, LOG_M_TOT - 2, LOG_M_TOT)
    base4 = lax.rem(p, 2) * M_PER
    step(_reader(src(), base4), M_PER, LOG_M_PER - 1, LOG_M_TOT)
    for j in range(LOG_M_PER - 2, -1, -1):
        step(_reader(src()), M_PER, j, LOG_M_TOT, last=(j == 0))


def _merge14(g):
    grid = N_COLS // _MERGE_CHUNK
    return pl.pallas_call(
        _merge14_body,
        grid=(grid,),
        in_specs=[pl.BlockSpec((M_TOT, _MERGE_CHUNK), lambda c: (0, c))],
        out_specs=pl.BlockSpec((M_PER, _MERGE_CHUNK), lambda c: (0, c)),
        out_shape=jax.ShapeDtypeStruct((M_PER, N_COLS), jnp.bfloat16),
        scratch_shapes=[
            pltpu.VMEM((M_TOT, _MERGE_CHUNK), jnp.bfloat16),
            pltpu.VMEM((M_TOT, _MERGE_CHUNK), jnp.bfloat16),
        ],
        compiler_params=_VMEM_LIMIT,
    )(g)


def kernel(x):
    xs = _local_sort(x)
    pb = _exchange(
        xs,
        peer_of=lambda p: p ^ 1,
        slot_of=lambda p: lax.rem(p, 2),
        collective_id=0,
    )
    mine = _pair_merge(pb)
    g = _exchange(
        mine,
        peer_of=lambda p: 3 - p,
        slot_of=lambda p: lax.div(p, 2),
        collective_id=1,
    )
    return _merge14(g)


# device time: 820300 ns/iter; 1.4479x vs baseline; 1.0881x over previous
import functools

import jax
import jax.numpy as jnp
from jax import lax
from jax.experimental import pallas as pl
from jax.experimental.pallas import tpu as pltpu

N_DEV = 4
LOG_M_PER = 12
M_PER = 1 << LOG_M_PER
LOG_M_TOT = 14
M_TOT = N_DEV * M_PER
N_COLS = 1024

_TILE = 2048
_SORT_CHUNK = 256
_MERGE_CHUNK = 128

_VMEM_LIMIT = pltpu.CompilerParams(vmem_limit_bytes=60 * 1024 * 1024)


_STATIC_MIN = 512


def _cx_tiles(read, write, m, j, k, desc_flip=None):
    d = 1 << j
    if d >= _STATIC_MIN:
        ts = min(_TILE, d)
        for t in range(0, m, ts):
            first = ((t >> j) & 1) == 0
            pt = t + d if first else t - d
            a = read(t, ts)
            b = read(pt, ts)
            desc = ((t >> k) & 1) == 1
            take_min = first != desc
            if desc_flip is None:
                r = jnp.minimum(a, b) if take_min else jnp.maximum(a, b)
            else:
                lo, hi = jnp.minimum(a, b), jnp.maximum(a, b)
                r = jnp.where(
                    jnp.logical_xor(take_min, desc_flip), lo, hi
                )
            write(t, r, ts)
    else:
        iota = lax.broadcasted_iota(jnp.int32, (_TILE, 1), 0)
        is_first = ((iota >> j) & 1) == 0
        uniform = (1 << k) >= _TILE
        for t in range(0, m, _TILE):
            x = read(t, _TILE)
            up = jnp.roll(x, -d, axis=0)
            down = jnp.roll(x, d, axis=0)
            if uniform and desc_flip is None:
                if ((t >> k) & 1) == 1:
                    r = jnp.where(is_first, jnp.maximum(x, up),
                                  jnp.minimum(x, down))
                else:
                    r = jnp.where(is_first, jnp.minimum(x, up),
                                  jnp.maximum(x, down))
            else:
                partner = jnp.where(is_first, up, down)
                if uniform:
                    desc = ((t >> k) & 1) == 1
                    take_min = jnp.logical_not(is_first) if desc else is_first
                else:
                    descv = ((iota >> k) & 1) == 1
                    take_min = jnp.logical_xor(is_first, descv)
                if desc_flip is not None:
                    take_min = jnp.logical_xor(take_min, desc_flip)
                r = jnp.where(
                    take_min, jnp.minimum(x, partner), jnp.maximum(x, partner)
                )
            write(t, r, _TILE)


def _reader(ref, off=None):
    if off is None:
        return lambda t, h: ref[pl.ds(t, h), :]
    return lambda t, h: ref[pl.ds(off + t, h), :]


def _writer(ref):
    def w(t, v, h):
        ref[pl.ds(t, h), :] = v
    return w


def _local_sort_body(x_ref, o_ref, s0, s1):
    p = lax.axis_index("i")
    odd = lax.rem(p, 2) == 1
    subs = [
        (j, k)
        for k in range(1, LOG_M_PER + 1)
        for j in range(k - 1, -1, -1)
    ]
    bufs = [s0, s1]
    for idx, (j, k) in enumerate(subs):
        if idx == 0:
            read = lambda t, h: x_ref[pl.ds(t, h), :].astype(jnp.bfloat16)
        else:
            read = _reader(bufs[(idx - 1) % 2])
        dst = o_ref if idx == len(subs) - 1 else bufs[idx % 2]
        flip = odd if k == LOG_M_PER else None
        _cx_tiles(read, _writer(dst), M_PER, j, k, desc_flip=flip)


def _local_sort(x):
    m, n = x.shape
    grid = n // _SORT_CHUNK
    return pl.pallas_call(
        _local_sort_body,
        grid=(grid,),
        in_specs=[pl.BlockSpec((m, _SORT_CHUNK), lambda c: (0, c))],
        out_specs=pl.BlockSpec((m, _SORT_CHUNK), lambda c: (0, c)),
        out_shape=jax.ShapeDtypeStruct((m, n), jnp.bfloat16),
        scratch_shapes=[
            pltpu.VMEM((M_PER, _SORT_CHUNK), jnp.bfloat16),
            pltpu.VMEM((M_PER, _SORT_CHUNK), jnp.bfloat16),
        ],
        compiler_params=_VMEM_LIMIT,
    )(x)


def _exchange_body(x_ref, o_ref, send_sem, recv_sem, copy_sem, *, peer_of,
                   slot_of):
    p = lax.axis_index("i")
    peer = peer_of(p)
    m = x_ref.shape[0]
    off = slot_of(p) * m

    barrier_sem = pltpu.get_barrier_semaphore()
    pl.semaphore_signal(
        barrier_sem, inc=1,
        device_id=(peer,), device_id_type=pl.DeviceIdType.MESH,
    )
    pl.semaphore_wait(barrier_sem, 1)

    rdma = pltpu.make_async_remote_copy(
        src_ref=x_ref,
        dst_ref=o_ref.at[pl.ds(off, m), :],
        send_sem=send_sem,
        recv_sem=recv_sem,
        device_id=(peer,),
        device_id_type=pl.DeviceIdType.MESH,
    )
    rdma.start()
    own = pltpu.make_async_copy(
        x_ref, o_ref.at[pl.ds(off, m), :], copy_sem
    )
    own.start()
    own.wait()
    rdma.wait()


def _exchange(x, peer_of, slot_of, collective_id):
    m, n = x.shape
    body = functools.partial(
        _exchange_body, peer_of=peer_of, slot_of=slot_of
    )
    return pl.pallas_call(
        body,
        in_specs=[pl.BlockSpec(memory_space=pltpu.VMEM)],
        out_specs=pl.BlockSpec(memory_space=pl.ANY),
        out_shape=jax.ShapeDtypeStruct((2 * m, n), jnp.bfloat16),
        scratch_shapes=[
            pltpu.SemaphoreType.DMA,
            pltpu.SemaphoreType.DMA,
            pltpu.SemaphoreType.DMA,
        ],
        compiler_params=pltpu.CompilerParams(collective_id=collective_id),
    )(x)


_MS_CHUNK = 128
_MS_GRID = N_COLS // _MS_CHUNK


def _pair_merge_send_body(pb_ref, g_ref, s0, s1, slots, send_sems,
                          recv_sems, copy_sem):
    p = lax.axis_index("i")
    opp = 3 - p
    pair_odd = lax.div(p, 2) == 1
    row_off = lax.div(p, 2) * (2 * M_PER)
    c = pl.program_id(0)

    @pl.when(c == 0)
    def _():
        barrier_sem = pltpu.get_barrier_semaphore()
        pl.semaphore_signal(
            barrier_sem, inc=1,
            device_id=(opp,), device_id_type=pl.DeviceIdType.MESH,
        )
        pl.semaphore_wait(barrier_sem, 1)

    def send_desc(send_sem, recv_sem, dst):
        return pltpu.make_async_remote_copy(
            src_ref=slots.at[lax.rem(c, 2)],
            dst_ref=dst,
            send_sem=send_sem,
            recv_sem=recv_sem,
            device_id=(opp,),
            device_id_type=pl.DeviceIdType.MESH,
        )

    @pl.when(c >= 2)
    def _():
        send_desc(send_sems.at[c - 2], recv_sems.at[0],
                  slots.at[lax.rem(c, 2)]).wait_send()

    slot = slots.at[lax.rem(c, 2)]
    bufs = [s0, s1]
    subs = list(range(LOG_M_PER, -1, -1))
    for idx, j in enumerate(subs):
        read = _reader(pb_ref if idx == 0 else bufs[(idx - 1) % 2])
        dst = slot if idx == len(subs) - 1 else bufs[idx % 2]
        _cx_tiles(read, _writer(dst), 2 * M_PER, j, LOG_M_TOT - 1,
                  desc_flip=pair_odd)

    dstg = g_ref.at[
        pl.ds(row_off, 2 * M_PER), pl.ds(c * _MS_CHUNK, _MS_CHUNK)
    ]
    own = pltpu.make_async_copy(slot, dstg, copy_sem)
    own.start()
    own.wait()
    send_desc(send_sems.at[c], recv_sems.at[c], dstg).start()

    @pl.when(c == _MS_GRID - 1)
    def _():
        for i in (_MS_GRID - 2, _MS_GRID - 1):
            pltpu.make_async_remote_copy(
                src_ref=slots.at[i % 2],
                dst_ref=slots.at[i % 2],
                send_sem=send_sems.at[i],
                recv_sem=recv_sems.at[0],
                device_id=(opp,),
                device_id_type=pl.DeviceIdType.MESH,
            ).wait_send()
        for i in range(_MS_GRID):
            pltpu.make_async_remote_copy(
                src_ref=slots.at[i % 2],
                dst_ref=g_ref.at[
                    pl.ds(0, 2 * M_PER), pl.ds(i * _MS_CHUNK, _MS_CHUNK)
                ],
                send_sem=send_sems.at[i],
                recv_sem=recv_sems.at[i],
                device_id=(opp,),
                device_id_type=pl.DeviceIdType.MESH,
            ).wait_recv()


def _pair_merge_send(pb):
    m = 2 * M_PER
    return pl.pallas_call(
        _pair_merge_send_body,
        grid=(_MS_GRID,),
        in_specs=[pl.BlockSpec((m, _MS_CHUNK), lambda c: (0, c))],
        out_specs=pl.BlockSpec(memory_space=pl.ANY),
        out_shape=jax.ShapeDtypeStruct((M_TOT, N_COLS), jnp.bfloat16),
        scratch_shapes=[
            pltpu.VMEM((m, _MS_CHUNK), jnp.bfloat16),
            pltpu.VMEM((m, _MS_CHUNK), jnp.bfloat16),
            pltpu.VMEM((2, m, _MS_CHUNK), jnp.bfloat16),
            pltpu.SemaphoreType.DMA((_MS_GRID,)),
            pltpu.SemaphoreType.DMA((_MS_GRID,)),
            pltpu.SemaphoreType.DMA,
        ],
        compiler_params=pltpu.CompilerParams(
            collective_id=1, vmem_limit_bytes=60 * 1024 * 1024
        ),
    )(pb)


def _merge14_body(g_ref, o_ref, s0, s1):
    p = lax.axis_index("i")
    bufs = [s0, s1]
    idx = 0

    def step(read, height, j, k, last=False):
        nonlocal idx
        dst = o_ref if last else bufs[idx % 2]
        _cx_tiles(read, _writer(dst), height, j, k)
        idx += 1

    def src():
        return bufs[(idx - 1) % 2]

    step(_reader(g_ref), M_TOT, LOG_M_TOT - 1, LOG_M_TOT)
    base8 = lax.div(p, 2) * (M_TOT // 2)
    step(_reader(src(), base8), M_TOT // 2, LOG_M_TOT - 2, LOG_M_TOT)
    base4 = lax.rem(p, 2) * M_PER
    step(_reader(src(), base4), M_PER, LOG_M_PER - 1, LOG_M_TOT)
    for j in range(LOG_M_PER - 2, -1, -1):
        step(_reader(src()), M_PER, j, LOG_M_TOT, last=(j == 0))


def _merge14(g):
    grid = N_COLS // _MERGE_CHUNK
    return pl.pallas_call(
        _merge14_body,
        grid=(grid,),
        in_specs=[pl.BlockSpec((M_TOT, _MERGE_CHUNK), lambda c: (0, c))],
        out_specs=pl.BlockSpec((M_PER, _MERGE_CHUNK), lambda c: (0, c)),
        out_shape=jax.ShapeDtypeStruct((M_PER, N_COLS), jnp.bfloat16),
        scratch_shapes=[
            pltpu.VMEM((M_TOT, _MERGE_CHUNK), jnp.bfloat16),
            pltpu.VMEM((M_TOT, _MERGE_CHUNK), jnp.bfloat16),
        ],
        compiler_params=_VMEM_LIMIT,
    )(g)


def kernel(x):
    xs = _local_sort(x)
    pb = _exchange(
        xs,
        peer_of=lambda p: p ^ 1,
        slot_of=lambda p: lax.rem(p, 2),
        collective_id=0,
    )
    g = _pair_merge_send(pb)
    return _merge14(g)


# device time: 744873 ns/iter; 1.5945x vs baseline; 1.1013x over previous
import functools

import jax
import jax.numpy as jnp
from jax import lax
from jax.experimental import pallas as pl
from jax.experimental.pallas import tpu as pltpu

N_DEV = 4
LOG_M_PER = 12
M_PER = 1 << LOG_M_PER
LOG_M_TOT = 14
M_TOT = N_DEV * M_PER
N_COLS = 1024

_TILE = 2048
_SORT_CHUNK = 256
_MERGE_CHUNK = 128

_VMEM_LIMIT = pltpu.CompilerParams(vmem_limit_bytes=60 * 1024 * 1024)


_STATIC_MIN = 512


def _cx_tiles(read, write, m, j, k, desc_flip=None):
    d = 1 << j
    if d >= _STATIC_MIN:
        ts = min(_TILE, d)
        for t in range(0, m, ts):
            first = ((t >> j) & 1) == 0
            pt = t + d if first else t - d
            a = read(t, ts)
            b = read(pt, ts)
            desc = ((t >> k) & 1) == 1
            take_min = first != desc
            if desc_flip is None:
                r = jnp.minimum(a, b) if take_min else jnp.maximum(a, b)
            else:
                lo, hi = jnp.minimum(a, b), jnp.maximum(a, b)
                r = jnp.where(
                    jnp.logical_xor(take_min, desc_flip), lo, hi
                )
            write(t, r, ts)
    else:
        iota = lax.broadcasted_iota(jnp.int32, (_TILE, 1), 0)
        is_first = ((iota >> j) & 1) == 0
        uniform = (1 << k) >= _TILE
        for t in range(0, m, _TILE):
            x = read(t, _TILE)
            up = jnp.roll(x, -d, axis=0)
            down = jnp.roll(x, d, axis=0)
            if uniform and desc_flip is None:
                if ((t >> k) & 1) == 1:
                    r = jnp.where(is_first, jnp.maximum(x, up),
                                  jnp.minimum(x, down))
                else:
                    r = jnp.where(is_first, jnp.minimum(x, up),
                                  jnp.maximum(x, down))
            else:
                partner = jnp.where(is_first, up, down)
                if uniform:
                    desc = ((t >> k) & 1) == 1
                    take_min = jnp.logical_not(is_first) if desc else is_first
                else:
                    descv = ((iota >> k) & 1) == 1
                    take_min = jnp.logical_xor(is_first, descv)
                if desc_flip is not None:
                    take_min = jnp.logical_xor(take_min, desc_flip)
                r = jnp.where(
                    take_min, jnp.minimum(x, partner), jnp.maximum(x, partner)
                )
            write(t, r, _TILE)


def _reader(ref, off=None):
    if off is None:
        return lambda t, h: ref[pl.ds(t, h), :]
    return lambda t, h: ref[pl.ds(off + t, h), :]


def _writer(ref):
    def w(t, v, h):
        ref[pl.ds(t, h), :] = v
    return w


_LS_GRID = N_COLS // _SORT_CHUNK


def _local_sort_send_body(x_ref, pb_ref, s0, s1, slots, send_sems,
                          recv_sems, copy_sem):
    p = lax.axis_index("i")
    partner = p ^ 1
    odd = lax.rem(p, 2) == 1
    row_off = lax.rem(p, 2) * M_PER
    c = pl.program_id(0)

    @pl.when(c == 0)
    def _():
        barrier_sem = pltpu.get_barrier_semaphore()
        pl.semaphore_signal(
            barrier_sem, inc=1,
            device_id=(partner,), device_id_type=pl.DeviceIdType.MESH,
        )
        pl.semaphore_wait(barrier_sem, 1)

    @pl.when(c >= 2)
    def _():
        pltpu.make_async_remote_copy(
            src_ref=slots.at[lax.rem(c, 2)],
            dst_ref=slots.at[lax.rem(c, 2)],
            send_sem=send_sems.at[c - 2],
            recv_sem=recv_sems.at[0],
            device_id=(partner,),
            device_id_type=pl.DeviceIdType.MESH,
        ).wait_send()

    slot = slots.at[lax.rem(c, 2)]
    subs = [
        (j, k)
        for k in range(1, LOG_M_PER + 1)
        for j in range(k - 1, -1, -1)
    ]
    bufs = [s0, s1]
    for idx, (j, k) in enumerate(subs):
        if idx == 0:
            read = lambda t, h: x_ref[pl.ds(t, h), :].astype(jnp.bfloat16)
        else:
            read = _reader(bufs[(idx - 1) % 2])
        dst = slot if idx == len(subs) - 1 else bufs[idx % 2]
        flip = odd if k == LOG_M_PER else None
        _cx_tiles(read, _writer(dst), M_PER, j, k, desc_flip=flip)

    dstg = pb_ref.at[
        pl.ds(row_off, M_PER), pl.ds(c * _SORT_CHUNK, _SORT_CHUNK)
    ]
    own = pltpu.make_async_copy(slot, dstg, copy_sem)
    own.start()
    own.wait()
    pltpu.make_async_remote_copy(
        src_ref=slot,
        dst_ref=dstg,
        send_sem=send_sems.at[c],
        recv_sem=recv_sems.at[c],
        device_id=(partner,),
        device_id_type=pl.DeviceIdType.MESH,
    ).start()

    @pl.when(c == _LS_GRID - 1)
    def _():
        for i in (_LS_GRID - 2, _LS_GRID - 1):
            pltpu.make_async_remote_copy(
                src_ref=slots.at[i % 2],
                dst_ref=slots.at[i % 2],
                send_sem=send_sems.at[i],
                recv_sem=recv_sems.at[0],
                device_id=(partner,),
                device_id_type=pl.DeviceIdType.MESH,
            ).wait_send()
        for i in range(_LS_GRID):
            pltpu.make_async_remote_copy(
                src_ref=slots.at[i % 2],
                dst_ref=pb_ref.at[
                    pl.ds(0, M_PER), pl.ds(i * _SORT_CHUNK, _SORT_CHUNK)
                ],
                send_sem=send_sems.at[i],
                recv_sem=recv_sems.at[i],
                device_id=(partner,),
                device_id_type=pl.DeviceIdType.MESH,
            ).wait_recv()


def _local_sort_send(x):
    m, n = x.shape
    return pl.pallas_call(
        _local_sort_send_body,
        grid=(_LS_GRID,),
        in_specs=[pl.BlockSpec((m, _SORT_CHUNK), lambda c: (0, c))],
        out_specs=pl.BlockSpec(memory_space=pl.ANY),
        out_shape=jax.ShapeDtypeStruct((2 * m, n), jnp.bfloat16),
        scratch_shapes=[
            pltpu.VMEM((M_PER, _SORT_CHUNK), jnp.bfloat16),
            pltpu.VMEM((M_PER, _SORT_CHUNK), jnp.bfloat16),
            pltpu.VMEM((2, M_PER, _SORT_CHUNK), jnp.bfloat16),
            pltpu.SemaphoreType.DMA((_LS_GRID,)),
            pltpu.SemaphoreType.DMA((_LS_GRID,)),
            pltpu.SemaphoreType.DMA,
        ],
        compiler_params=pltpu.CompilerParams(
            collective_id=0, vmem_limit_bytes=60 * 1024 * 1024
        ),
    )(x)


def _exchange_body(x_ref, o_ref, send_sem, recv_sem, copy_sem, *, peer_of,
                   slot_of):
    p = lax.axis_index("i")
    peer = peer_of(p)
    m = x_ref.shape[0]
    off = slot_of(p) * m

    barrier_sem = pltpu.get_barrier_semaphore()
    pl.semaphore_signal(
        barrier_sem, inc=1,
        device_id=(peer,), device_id_type=pl.DeviceIdType.MESH,
    )
    pl.semaphore_wait(barrier_sem, 1)

    rdma = pltpu.make_async_remote_copy(
        src_ref=x_ref,
        dst_ref=o_ref.at[pl.ds(off, m), :],
        send_sem=send_sem,
        recv_sem=recv_sem,
        device_id=(peer,),
        device_id_type=pl.DeviceIdType.MESH,
    )
    rdma.start()
    own = pltpu.make_async_copy(
        x_ref, o_ref.at[pl.ds(off, m), :], copy_sem
    )
    own.start()
    own.wait()
    rdma.wait()


def _exchange(x, peer_of, slot_of, collective_id):
    m, n = x.shape
    body = functools.partial(
        _exchange_body, peer_of=peer_of, slot_of=slot_of
    )
    return pl.pallas_call(
        body,
        in_specs=[pl.BlockSpec(memory_space=pltpu.VMEM)],
        out_specs=pl.BlockSpec(memory_space=pl.ANY),
        out_shape=jax.ShapeDtypeStruct((2 * m, n), jnp.bfloat16),
        scratch_shapes=[
            pltpu.SemaphoreType.DMA,
            pltpu.SemaphoreType.DMA,
            pltpu.SemaphoreType.DMA,
        ],
        compiler_params=pltpu.CompilerParams(collective_id=collective_id),
    )(x)


_MS_CHUNK = 128
_MS_GRID = N_COLS // _MS_CHUNK


def _pair_merge_send_body(pb_ref, g_ref, s0, s1, slots, send_sems,
                          recv_sems, copy_sem):
    p = lax.axis_index("i")
    opp = 3 - p
    pair_odd = lax.div(p, 2) == 1
    row_off = lax.div(p, 2) * (2 * M_PER)
    c = pl.program_id(0)

    @pl.when(c == 0)
    def _():
        barrier_sem = pltpu.get_barrier_semaphore()
        pl.semaphore_signal(
            barrier_sem, inc=1,
            device_id=(opp,), device_id_type=pl.DeviceIdType.MESH,
        )
        pl.semaphore_wait(barrier_sem, 1)

    def send_desc(send_sem, recv_sem, dst):
        return pltpu.make_async_remote_copy(
            src_ref=slots.at[lax.rem(c, 2)],
            dst_ref=dst,
            send_sem=send_sem,
            recv_sem=recv_sem,
            device_id=(opp,),
            device_id_type=pl.DeviceIdType.MESH,
        )

    @pl.when(c >= 2)
    def _():
        send_desc(send_sems.at[c - 2], recv_sems.at[0],
                  slots.at[lax.rem(c, 2)]).wait_send()

    slot = slots.at[lax.rem(c, 2)]
    bufs = [s0, s1]
    subs = list(range(LOG_M_PER, -1, -1))
    for idx, j in enumerate(subs):
        read = _reader(pb_ref if idx == 0 else bufs[(idx - 1) % 2])
        dst = slot if idx == len(subs) - 1 else bufs[idx % 2]
        _cx_tiles(read, _writer(dst), 2 * M_PER, j, LOG_M_TOT - 1,
                  desc_flip=pair_odd)

    dstg = g_ref.at[
        pl.ds(row_off, 2 * M_PER), pl.ds(c * _MS_CHUNK, _MS_CHUNK)
    ]
    own = pltpu.make_async_copy(slot, dstg, copy_sem)
    own.start()
    own.wait()
    send_desc(send_sems.at[c], recv_sems.at[c], dstg).start()

    @pl.when(c == _MS_GRID - 1)
    def _():
        for i in (_MS_GRID - 2, _MS_GRID - 1):
            pltpu.make_async_remote_copy(
                src_ref=slots.at[i % 2],
                dst_ref=slots.at[i % 2],
                send_sem=send_sems.at[i],
                recv_sem=recv_sems.at[0],
                device_id=(opp,),
                device_id_type=pl.DeviceIdType.MESH,
            ).wait_send()
        for i in range(_MS_GRID):
            pltpu.make_async_remote_copy(
                src_ref=slots.at[i % 2],
                dst_ref=g_ref.at[
                    pl.ds(0, 2 * M_PER), pl.ds(i * _MS_CHUNK, _MS_CHUNK)
                ],
                send_sem=send_sems.at[i],
                recv_sem=recv_sems.at[i],
                device_id=(opp,),
                device_id_type=pl.DeviceIdType.MESH,
            ).wait_recv()


def _pair_merge_send(pb):
    m = 2 * M_PER
    return pl.pallas_call(
        _pair_merge_send_body,
        grid=(_MS_GRID,),
        in_specs=[pl.BlockSpec((m, _MS_CHUNK), lambda c: (0, c))],
        out_specs=pl.BlockSpec(memory_space=pl.ANY),
        out_shape=jax.ShapeDtypeStruct((M_TOT, N_COLS), jnp.bfloat16),
        scratch_shapes=[
            pltpu.VMEM((m, _MS_CHUNK), jnp.bfloat16),
            pltpu.VMEM((m, _MS_CHUNK), jnp.bfloat16),
            pltpu.VMEM((2, m, _MS_CHUNK), jnp.bfloat16),
            pltpu.SemaphoreType.DMA((_MS_GRID,)),
            pltpu.SemaphoreType.DMA((_MS_GRID,)),
            pltpu.SemaphoreType.DMA,
        ],
        compiler_params=pltpu.CompilerParams(
            collective_id=1, vmem_limit_bytes=60 * 1024 * 1024
        ),
    )(pb)


def _merge14_body(g_ref, o_ref, s0, s1):
    p = lax.axis_index("i")
    bufs = [s0, s1]
    idx = 0

    def step(read, height, j, k, last=False):
        nonlocal idx
        dst = o_ref if last else bufs[idx % 2]
        _cx_tiles(read, _writer(dst), height, j, k)
        idx += 1

    def src():
        return bufs[(idx - 1) % 2]

    step(_reader(g_ref), M_TOT, LOG_M_TOT - 1, LOG_M_TOT)
    base8 = lax.div(p, 2) * (M_TOT // 2)
    step(_reader(src(), base8), M_TOT // 2, LOG_M_TOT - 2, LOG_M_TOT)
    base4 = lax.rem(p, 2) * M_PER
    step(_reader(src(), base4), M_PER, LOG_M_PER - 1, LOG_M_TOT)
    for j in range(LOG_M_PER - 2, -1, -1):
        step(_reader(src()), M_PER, j, LOG_M_TOT, last=(j == 0))


def _merge14(g):
    grid = N_COLS // _MERGE_CHUNK
    return pl.pallas_call(
        _merge14_body,
        grid=(grid,),
        in_specs=[pl.BlockSpec((M_TOT, _MERGE_CHUNK), lambda c: (0, c))],
        out_specs=pl.BlockSpec((M_PER, _MERGE_CHUNK), lambda c: (0, c)),
        out_shape=jax.ShapeDtypeStruct((M_PER, N_COLS), jnp.bfloat16),
        scratch_shapes=[
            pltpu.VMEM((M_TOT, _MERGE_CHUNK), jnp.bfloat16),
            pltpu.VMEM((M_TOT, _MERGE_CHUNK), jnp.bfloat16),
        ],
        compiler_params=_VMEM_LIMIT,
    )(g)


def kernel(x):
    pb = _local_sort_send(x)
    g = _pair_merge_send(pb)
    return _merge14(g)
